# Initial kernel scaffold; baseline (speedup 1.0000x reference)
import jax
import jax.numpy as jnp
from jax.experimental import pallas as pl


def kernel(Xt, y_true):
    raise NotImplementedError("write your pallas kernel here")


# speedup vs baseline: 14.5021x; 14.5021x over previous
import jax
import jax.numpy as jnp
from jax.experimental import pallas as pl
from jax.experimental.pallas import tpu as pltpu

_N = 320000
_D = 128
_K = 128
_B = 1600


def _accum_body(y_ref, x_ref, sums_ref, sqs_ref, cnt_ref):
    step = pl.program_id(0)

    @pl.when(step == 0)
    def _init():
        sums_ref[...] = jnp.zeros_like(sums_ref)
        sqs_ref[...] = jnp.zeros_like(sqs_ref)
        cnt_ref[...] = jnp.zeros_like(cnt_ref)

    x = x_ref[...]
    y = y_ref[0]
    klane = jax.lax.broadcasted_iota(jnp.int32, (_K, _B), 0)
    onehot_t = (klane == y).astype(jnp.float32)
    sums_ref[...] += jax.lax.dot(
        onehot_t, x, preferred_element_type=jnp.float32
    )
    sqs_ref[...] += jax.lax.dot(
        onehot_t, x * x, preferred_element_type=jnp.float32
    )
    cnt_ref[...] += jnp.sum(onehot_t, axis=1, keepdims=True)


def _finish_body(sums_ref, sqs_ref, cnt_ref, out_ref):
    sums = sums_ref[...]
    sqs = sqs_ref[...]
    cnt = cnt_ref[...]
    safe = jnp.maximum(cnt, 1.0)
    sumsq_k = jnp.sum(sqs, axis=1, keepdims=True)
    s_norm2 = jnp.sum(sums * sums, axis=1, keepdims=True)
    sq_per_class = sumsq_k - s_norm2 / safe
    per_class_mse = jnp.where(cnt > 0, sq_per_class / (safe * _D), 0.0)
    num_present = jnp.sum((cnt > 0).astype(jnp.float32))
    out_ref[0, 0] = jnp.sum(per_class_mse) / num_present


def kernel(Xt, y_true):
    grid = _N // _B
    y3 = y_true.reshape(grid, 1, _B)
    sums, sqs, cnt = pl.pallas_call(
        _accum_body,
        grid=(grid,),
        in_specs=[
            pl.BlockSpec((1, 1, _B), lambda i: (i, 0, 0)),
            pl.BlockSpec((_B, _D), lambda i: (i, 0)),
        ],
        out_specs=[
            pl.BlockSpec((_K, _D), lambda i: (0, 0)),
            pl.BlockSpec((_K, _D), lambda i: (0, 0)),
            pl.BlockSpec((_K, 1), lambda i: (0, 0)),
        ],
        out_shape=[
            jax.ShapeDtypeStruct((_K, _D), jnp.float32),
            jax.ShapeDtypeStruct((_K, _D), jnp.float32),
            jax.ShapeDtypeStruct((_K, 1), jnp.float32),
        ],
    )(y3, Xt)

    loss = pl.pallas_call(
        _finish_body,
        out_specs=pl.BlockSpec(memory_space=pltpu.SMEM),
        out_shape=jax.ShapeDtypeStruct((1, 1), jnp.float32),
    )(sums, sqs, cnt)
    return loss[0, 0]
